# baseline (speedup 1.0000x reference)
import functools

import jax
import jax.numpy as jnp
import numpy as np
from jax import lax
from jax.experimental import pallas as pl
from jax.experimental.pallas import tpu as pltpu
from jax.experimental.pallas import tpu_sc as plsc

_MIN_KEPT = 100000
_B, _H, _W = 16, 512, 512
_N = _B * _H * _W
_INT_MAX = np.int32(2**31 - 1)

_HB = 512
_GRID = (_B, _H // _HB)

_NW = 32
_PER_TILE = _N // _NW
_CHUNK = 16384
_NCHUNK = _PER_TILE // _CHUNK
_HBINS = 4096
_LANES = 16
_UNROLL = 16


def _stage_a_body(pred_ref, tgt_ref, v_ref, npos_ref, lpos_ref):
    step = pl.program_id(0) * pl.num_programs(1) + pl.program_id(1)
    d = pred_ref[0, 0, :, :] - pred_ref[0, 1, :, :]
    t = tgt_ref[0]
    pos = t == 1
    th = jnp.tanh(d * 0.5)
    ce_pos = jnp.log1p(jnp.exp(th))
    lpos_part = jnp.sum(jnp.where(pos, ce_pos, jnp.float32(0.0)))
    npos_part = jnp.sum(t).astype(jnp.float32)
    b = lax.bitcast_convert_type(d, jnp.int32)
    v = jnp.where(b >= 0, b, b ^ jnp.int32(0x7FFFFFFF))
    v = jnp.where(pos, _INT_MAX, v)
    v_ref[...] = v

    @pl.when(step == 0)
    def _():
        npos_ref[0, 0] = jnp.float32(0.0)
        lpos_ref[0, 0] = jnp.float32(0.0)

    npos_ref[0, 0] += npos_part
    lpos_ref[0, 0] += lpos_part


_NBH = _B // 2


def _stage_a(predict, target, half):
    return pl.pallas_call(
        _stage_a_body,
        grid=(_NBH, 1),
        in_specs=[
            pl.BlockSpec((1, 2, _HB, _W),
                         lambda i, j: (i + half * _NBH, 0, j, 0)),
            pl.BlockSpec((1, _HB, _W), lambda i, j: (i + half * _NBH, j, 0)),
        ],
        out_specs=[
            pl.BlockSpec((_HB, _W), lambda i, j: (i * (_H // _HB) + j, 0)),
            pl.BlockSpec(memory_space=pltpu.SMEM),
            pl.BlockSpec(memory_space=pltpu.SMEM),
        ],
        out_shape=[
            jax.ShapeDtypeStruct((_NBH * _H, _W), jnp.int32),
            jax.ShapeDtypeStruct((1, 1), jnp.float32),
            jax.ShapeDtypeStruct((1, 1), jnp.float32),
        ],
        compiler_params=pltpu.CompilerParams(
            dimension_semantics=("arbitrary", "arbitrary")),
    )(predict, target)


_VROWS = _B * _H
_TROWS = _VROWS // _NW
_CROWS = _CHUNK // _W
_WGRP = _W // _LANES


def _hist_common(v_list, out_hbm, bufs, sems, hist, bin_fn, init_hbm=None):
    wid = lax.axis_index("s") * 2 + lax.axis_index("c")
    zeros = jnp.zeros((_LANES,), jnp.int32)
    ones = jnp.full((_LANES,), 1, jnp.int32)
    lanes = lax.iota(jnp.int32, _LANES)

    if init_hbm is not None:
        pltpu.sync_copy(init_hbm.at[wid], hist)
    else:
        def zbody(i, _):
            for u in range(_UNROLL):
                hist[pl.ds((i * _UNROLL + u) * _LANES, _LANES)] = zeros
            return 0

        lax.fori_loop(0, _HBINS // _UNROLL, zbody, 0)

    chunks = []
    for vref in v_list:
        trows = vref.shape[0] // _NW
        base_row = wid * trows
        for c in range(trows // _CROWS):
            chunks.append((vref, base_row + c * _CROWS))

    def src(i):
        vref, row0 = chunks[i]
        return vref.at[pl.ds(row0, _CROWS), :]

    nchunk = len(chunks)
    grp_per_j = _WGRP // _UNROLL
    pending = pltpu.async_copy(src(0), bufs[0], sems[0])
    for c in range(nchunk):
        slot = c % 2
        nxt = None
        if c + 1 < nchunk:
            nxt = pltpu.async_copy(src(c + 1), bufs[(c + 1) % 2],
                                   sems[(c + 1) % 2])
        pending.wait()
        buf = bufs[slot]

        def ibody(j, _):
            r = j // grp_per_j
            c0 = (j % grp_per_j) * _UNROLL
            xs = [buf[r, pl.ds((c0 + u) * _LANES, _LANES)]
                  for u in range(_UNROLL)]
            pairs = [bin_fn(x) for x in xs]
            idxs = [bn * _LANES + lanes for bn, _ in pairs]
            for (_, msk), idx in zip(pairs, idxs):
                plsc.addupdate_scatter(hist, [idx], ones, mask=msk)
            return 0

        lax.fori_loop(0, _CROWS * grp_per_j, ibody, 0)
        pending = nxt
    pltpu.sync_copy(hist, out_hbm.at[wid])


@functools.lru_cache(maxsize=None)
def _build_hist_kernels():
    mesh = plsc.VectorSubcoreMesh(core_axis_name="c", subcore_axis_name="s")

    @functools.partial(
        pl.kernel,
        mesh=mesh,
        out_type=jax.ShapeDtypeStruct((_NW, _HBINS * _LANES), jnp.int32),
        scratch_types=[
            pltpu.VMEM((_CROWS, _W), jnp.int32),
            pltpu.VMEM((_CROWS, _W), jnp.int32),
            pltpu.SemaphoreType.DMA,
            pltpu.SemaphoreType.DMA,
            pltpu.VMEM((_HBINS * _LANES,), jnp.int32),
        ],
        compiler_params=pltpu.CompilerParams(needs_layout_passes=False),
    )
    def _hist1(v_hbm, out_hbm, buf0, buf1, sem0, sem1, hist):
        def bin_fn(x):
            return (x >> 20) + 2048, jnp.full((_LANES,), True, jnp.bool_)

        _hist_common([v_hbm], out_hbm, (buf0, buf1), (sem0, sem1), hist,
                     bin_fn)

    @functools.partial(
        pl.kernel,
        mesh=mesh,
        out_type=jax.ShapeDtypeStruct((_NW, _HBINS * _LANES), jnp.int32),
        scratch_types=[
            pltpu.VMEM((_CROWS, _W), jnp.int32),
            pltpu.VMEM((_CROWS, _W), jnp.int32),
            pltpu.SemaphoreType.DMA,
            pltpu.SemaphoreType.DMA,
            pltpu.VMEM((_HBINS * _LANES,), jnp.int32),
        ],
        compiler_params=pltpu.CompilerParams(needs_layout_passes=False),
    )
    def _hist1b(v_hbm, init_hbm, out_hbm, buf0, buf1, sem0, sem1, hist):
        def bin_fn(x):
            return (x >> 20) + 2048, jnp.full((_LANES,), True, jnp.bool_)

        _hist_common([v_hbm], out_hbm, (buf0, buf1), (sem0, sem1), hist,
                     bin_fn, init_hbm=init_hbm)

    @functools.partial(
        pl.kernel,
        mesh=mesh,
        out_type=jax.ShapeDtypeStruct((_NW, _HBINS * _LANES), jnp.int32),
        scratch_types=[
            pltpu.VMEM((_CROWS, _W), jnp.int32),
            pltpu.VMEM((_CROWS, _W), jnp.int32),
            pltpu.SemaphoreType.DMA,
            pltpu.SemaphoreType.DMA,
            pltpu.VMEM((_HBINS * _LANES,), jnp.int32),
            pltpu.VMEM((_LANES,), jnp.int32),
        ],
        compiler_params=pltpu.CompilerParams(needs_layout_passes=False),
    )
    def _hist2(vlo_hbm, vhi_hbm, sel_hbm, out_hbm, buf0, buf1, sem0, sem1,
               hist, selbuf):
        pltpu.sync_copy(sel_hbm, selbuf)
        b1 = selbuf[pl.ds(0, _LANES)][0]

        def bin_fn(x):
            y = x >> 8
            msk = (y >> 12) == b1
            return y & 0xFFF, msk

        _hist_common([vlo_hbm, vhi_hbm], out_hbm, (buf0, buf1), (sem0, sem1),
                     hist, bin_fn)

    return _hist1, _hist1b, _hist2


def _stage_c_body(vt_ref, vlo_ref, vhi_ref, out_ref):
    step = pl.program_id(0) * pl.num_programs(1) + pl.program_id(1)
    part = jnp.float32(0.0)
    for v_ref in (vlo_ref, vhi_ref):
        v = v_ref[...]
        b = jnp.where(v >= 0, v, v ^ jnp.int32(0x7FFFFFFF))
        d = lax.bitcast_convert_type(b, jnp.float32)
        ce = jnp.log1p(jnp.exp(-jnp.tanh(d * 0.5)))
        sel = v < vt_ref[0, 0]
        part += jnp.sum(jnp.where(sel, ce, jnp.float32(0.0)))

    @pl.when(step == 0)
    def _():
        out_ref[0, 0] = jnp.float32(0.0)

    out_ref[0, 0] += part


def _stage_c(vt, v_lo, v_hi):
    return pl.pallas_call(
        _stage_c_body,
        grid=(_NBH, 1),
        in_specs=[
            pl.BlockSpec(memory_space=pltpu.SMEM),
            pl.BlockSpec((_HB, _W), lambda i, j: (i * (_H // _HB) + j, 0)),
            pl.BlockSpec((_HB, _W), lambda i, j: (i * (_H // _HB) + j, 0)),
        ],
        out_specs=pl.BlockSpec(memory_space=pltpu.SMEM),
        out_shape=jax.ShapeDtypeStruct((1, 1), jnp.float32),
        compiler_params=pltpu.CompilerParams(
            dimension_semantics=("arbitrary", "arbitrary")),
    )(vt, v_lo, v_hi)


def _ce_of_v(v):
    b = jnp.where(v >= 0, v, v ^ jnp.int32(0x7FFFFFFF))
    d = lax.bitcast_convert_type(b, jnp.float32)
    return jnp.log1p(jnp.exp(-jnp.tanh(d * 0.5)))


def kernel(predict, target):
    t32 = target.astype(jnp.int32)
    v_lo, npos_a, lpos_a = _stage_a(predict, t32, 0)
    v_hi, npos_b, lpos_b = _stage_a(predict, t32, 1)
    npos_f = npos_a[0, 0] + npos_b[0, 0]
    loss_pos = lpos_a[0, 0] + lpos_b[0, 0]
    npos = npos_f.astype(jnp.int32)
    neg_count = jnp.int32(_N) - npos
    n_neg = jnp.where(npos > 0,
                      jnp.minimum(neg_count, jnp.int32(_MIN_KEPT)),
                      jnp.int32(100))
    k_eff = jnp.minimum(n_neg, neg_count)

    _hist1, _hist1b, _hist2 = _build_hist_kernels()
    h1a = _hist1(v_lo)
    h1 = _hist1b(v_hi, h1a)
    h1 = h1.reshape(_NW, _HBINS, _LANES).sum(axis=(0, 2))
    c1 = jnp.cumsum(h1)
    b1 = jnp.sum((c1 < k_eff).astype(jnp.int32))
    c0 = jnp.where(b1 > 0, c1[jnp.maximum(b1 - 1, 0)], 0)
    r1 = k_eff - c0

    sel = jnp.zeros((_LANES,), jnp.int32).at[0].set(b1 - 2048)
    h2 = _hist2(v_lo, v_hi, sel)
    h2 = jnp.sum(h2.reshape(_NW, _HBINS, _LANES), axis=(0, 2))
    c2 = jnp.cumsum(h2)
    b2 = jnp.sum((c2 < r1).astype(jnp.int32))
    c_lt2 = jnp.where(b2 > 0, c2[jnp.maximum(b2 - 1, 0)], 0)
    r_ties = r1 - c_lt2

    tau24 = ((b1 - 2048) << 12) | b2
    vt = tau24 << 8
    v_rep = vt | 128

    sum_lt = _stage_c(jnp.full((1, 1), vt, jnp.int32), v_lo, v_hi)[0, 0]
    loss_neg = sum_lt + r_ties.astype(jnp.float32) * _ce_of_v(v_rep)
    loss_neg = jnp.where(k_eff > 0, loss_neg, jnp.float32(0.0))

    return (loss_pos + loss_neg) / (npos_f + n_neg.astype(jnp.float32))


# speedup vs baseline: 1.0126x; 1.0126x over previous
import functools

import jax
import jax.numpy as jnp
import numpy as np
from jax import lax
from jax.experimental import pallas as pl
from jax.experimental.pallas import tpu as pltpu
from jax.experimental.pallas import tpu_sc as plsc

_MIN_KEPT = 100000
_B, _H, _W = 16, 512, 512
_N = _B * _H * _W
_INT_MAX = np.int32(2**31 - 1)

_HB = 512
_GRID = (_B, _H // _HB)

_NW = 32
_PER_TILE = _N // _NW
_CHUNK = 16384
_NCHUNK = _PER_TILE // _CHUNK
_HBINS = 4096
_LANES = 16
_UNROLL = 16


def _stage_a_body(pred_ref, tgt_ref, v_ref, npos_ref, lpos_ref):
    step = pl.program_id(0) * pl.num_programs(1) + pl.program_id(1)
    d = pred_ref[0, 0, :, :] - pred_ref[0, 1, :, :]
    t = tgt_ref[0]
    pos = t == 1
    th = jnp.tanh(d * 0.5)
    ce_pos = jnp.log1p(jnp.exp(th))
    lpos_part = jnp.sum(jnp.where(pos, ce_pos, jnp.float32(0.0)))
    npos_part = jnp.sum(t).astype(jnp.float32)
    b = lax.bitcast_convert_type(d, jnp.int32)
    v = jnp.where(b >= 0, b, b ^ jnp.int32(0x7FFFFFFF))
    v = jnp.where(pos, _INT_MAX, v)
    v_ref[...] = v

    @pl.when(step == 0)
    def _():
        npos_ref[0, 0] = jnp.float32(0.0)
        lpos_ref[0, 0] = jnp.float32(0.0)

    npos_ref[0, 0] += npos_part
    lpos_ref[0, 0] += lpos_part


_NBH = _B // 2


def _stage_a(predict, target, half):
    return pl.pallas_call(
        _stage_a_body,
        grid=(_NBH, 1),
        in_specs=[
            pl.BlockSpec((1, 2, _HB, _W),
                         lambda i, j: (i + half * _NBH, 0, j, 0)),
            pl.BlockSpec((1, _HB, _W), lambda i, j: (i + half * _NBH, j, 0)),
        ],
        out_specs=[
            pl.BlockSpec((_HB, _W), lambda i, j: (i * (_H // _HB) + j, 0)),
            pl.BlockSpec(memory_space=pltpu.SMEM),
            pl.BlockSpec(memory_space=pltpu.SMEM),
        ],
        out_shape=[
            jax.ShapeDtypeStruct((_NBH * _H, _W), jnp.int32),
            jax.ShapeDtypeStruct((1, 1), jnp.float32),
            jax.ShapeDtypeStruct((1, 1), jnp.float32),
        ],
        compiler_params=pltpu.CompilerParams(
            dimension_semantics=("arbitrary", "arbitrary")),
    )(predict, target)


_VROWS = _B * _H
_TROWS = _VROWS // _NW
_CROWS = _CHUNK // _W
_WGRP = _W // _LANES


def _hist_common(v_list, out_hbm, bufs, sems, hist, bin_fn):
    wid = lax.axis_index("s") * 2 + lax.axis_index("c")
    zeros = jnp.zeros((_LANES,), jnp.int32)
    ones = jnp.full((_LANES,), 1, jnp.int32)
    lanes = lax.iota(jnp.int32, _LANES)

    def zbody(i, _):
        for u in range(_UNROLL):
            hist[pl.ds((i * _UNROLL + u) * _LANES, _LANES)] = zeros
        return 0

    lax.fori_loop(0, _HBINS // _UNROLL, zbody, 0)

    chunks = []
    for vref in v_list:
        trows = vref.shape[0] // _NW
        base_row = wid * trows
        for c in range(trows // _CROWS):
            chunks.append((vref, base_row + c * _CROWS))

    def src(i):
        vref, row0 = chunks[i]
        return vref.at[pl.ds(row0, _CROWS), :]

    nchunk = len(chunks)
    grp_per_j = _WGRP // _UNROLL
    pending = pltpu.async_copy(src(0), bufs[0], sems[0])
    for c in range(nchunk):
        slot = c % 2
        nxt = None
        if c + 1 < nchunk:
            nxt = pltpu.async_copy(src(c + 1), bufs[(c + 1) % 2],
                                   sems[(c + 1) % 2])
        pending.wait()
        buf = bufs[slot]

        def ibody(j, _):
            r = j // grp_per_j
            c0 = (j % grp_per_j) * _UNROLL
            xs = [buf[r, pl.ds((c0 + u) * _LANES, _LANES)]
                  for u in range(_UNROLL)]
            pairs = [bin_fn(x) for x in xs]
            idxs = [bn * _LANES + lanes for bn, _ in pairs]
            for (_, msk), idx in zip(pairs, idxs):
                plsc.addupdate_scatter(hist, [idx], ones, mask=msk)
            return 0

        lax.fori_loop(0, _CROWS * grp_per_j, ibody, 0)
        pending = nxt
    pltpu.sync_copy(hist, out_hbm.at[wid])


@functools.lru_cache(maxsize=None)
def _build_hist_kernels():
    mesh = plsc.VectorSubcoreMesh(core_axis_name="c", subcore_axis_name="s")

    @functools.partial(
        pl.kernel,
        mesh=mesh,
        out_type=jax.ShapeDtypeStruct((_NW, _HBINS * _LANES), jnp.int32),
        scratch_types=[
            pltpu.VMEM((_CROWS, _W), jnp.int32),
            pltpu.VMEM((_CROWS, _W), jnp.int32),
            pltpu.SemaphoreType.DMA,
            pltpu.SemaphoreType.DMA,
            pltpu.VMEM((_HBINS * _LANES,), jnp.int32),
        ],
        compiler_params=pltpu.CompilerParams(needs_layout_passes=False),
    )
    def _hist1(v_hbm, out_hbm, buf0, buf1, sem0, sem1, hist):
        def bin_fn(x):
            return (x >> 20) + 2048, jnp.full((_LANES,), True, jnp.bool_)

        _hist_common([v_hbm], out_hbm, (buf0, buf1), (sem0, sem1), hist,
                     bin_fn)

    @functools.partial(
        pl.kernel,
        mesh=mesh,
        out_type=jax.ShapeDtypeStruct((_NW, _HBINS * _LANES), jnp.int32),
        scratch_types=[
            pltpu.VMEM((_CROWS, _W), jnp.int32),
            pltpu.VMEM((_CROWS, _W), jnp.int32),
            pltpu.SemaphoreType.DMA,
            pltpu.SemaphoreType.DMA,
            pltpu.VMEM((_HBINS * _LANES,), jnp.int32),
            pltpu.VMEM((_LANES,), jnp.int32),
        ],
        compiler_params=pltpu.CompilerParams(needs_layout_passes=False),
    )
    def _hist2(vlo_hbm, vhi_hbm, sel_hbm, out_hbm, buf0, buf1, sem0, sem1,
               hist, selbuf):
        pltpu.sync_copy(sel_hbm, selbuf)
        b1 = selbuf[pl.ds(0, _LANES)][0]

        def bin_fn(x):
            y = x >> 8
            msk = (y >> 12) == b1
            return y & 0xFFF, msk

        _hist_common([vlo_hbm, vhi_hbm], out_hbm, (buf0, buf1), (sem0, sem1),
                     hist, bin_fn)

    return _hist1, _hist2


def _stage_c_body(vt_ref, vlo_ref, vhi_ref, out_ref):
    step = pl.program_id(0) * pl.num_programs(1) + pl.program_id(1)
    part = jnp.float32(0.0)
    for v_ref in (vlo_ref, vhi_ref):
        v = v_ref[...]
        b = jnp.where(v >= 0, v, v ^ jnp.int32(0x7FFFFFFF))
        d = lax.bitcast_convert_type(b, jnp.float32)
        ce = jnp.log1p(jnp.exp(-jnp.tanh(d * 0.5)))
        sel = v < vt_ref[0, 0]
        part += jnp.sum(jnp.where(sel, ce, jnp.float32(0.0)))

    @pl.when(step == 0)
    def _():
        out_ref[0, 0] = jnp.float32(0.0)

    out_ref[0, 0] += part


def _stage_c(vt, v_lo, v_hi):
    return pl.pallas_call(
        _stage_c_body,
        grid=(_NBH, 1),
        in_specs=[
            pl.BlockSpec(memory_space=pltpu.SMEM),
            pl.BlockSpec((_HB, _W), lambda i, j: (i * (_H // _HB) + j, 0)),
            pl.BlockSpec((_HB, _W), lambda i, j: (i * (_H // _HB) + j, 0)),
        ],
        out_specs=pl.BlockSpec(memory_space=pltpu.SMEM),
        out_shape=jax.ShapeDtypeStruct((1, 1), jnp.float32),
        compiler_params=pltpu.CompilerParams(
            dimension_semantics=("arbitrary", "arbitrary")),
    )(vt, v_lo, v_hi)


def _ce_of_v(v):
    b = jnp.where(v >= 0, v, v ^ jnp.int32(0x7FFFFFFF))
    d = lax.bitcast_convert_type(b, jnp.float32)
    return jnp.log1p(jnp.exp(-jnp.tanh(d * 0.5)))


def kernel(predict, target):
    t32 = target.astype(jnp.int32)
    v_lo, npos_a, lpos_a = _stage_a(predict, t32, 0)
    v_hi, npos_b, lpos_b = _stage_a(predict, t32, 1)
    npos_f = npos_a[0, 0] + npos_b[0, 0]
    loss_pos = lpos_a[0, 0] + lpos_b[0, 0]
    npos = npos_f.astype(jnp.int32)
    neg_count = jnp.int32(_N) - npos
    n_neg = jnp.where(npos > 0,
                      jnp.minimum(neg_count, jnp.int32(_MIN_KEPT)),
                      jnp.int32(100))
    k_eff = jnp.minimum(n_neg, neg_count)

    _hist1, _hist2 = _build_hist_kernels()
    h1a = _hist1(v_lo)
    h1b = _hist1(v_hi)
    h1 = (h1a.reshape(_NW, _HBINS, _LANES).sum(axis=(0, 2))
          + h1b.reshape(_NW, _HBINS, _LANES).sum(axis=(0, 2)))
    c1 = jnp.cumsum(h1)
    b1 = jnp.sum((c1 < k_eff).astype(jnp.int32))
    c0 = jnp.where(b1 > 0, c1[jnp.maximum(b1 - 1, 0)], 0)
    r1 = k_eff - c0

    sel = jnp.zeros((_LANES,), jnp.int32).at[0].set(b1 - 2048)
    h2 = _hist2(v_lo, v_hi, sel)
    h2 = jnp.sum(h2.reshape(_NW, _HBINS, _LANES), axis=(0, 2))
    c2 = jnp.cumsum(h2)
    b2 = jnp.sum((c2 < r1).astype(jnp.int32))
    c_lt2 = jnp.where(b2 > 0, c2[jnp.maximum(b2 - 1, 0)], 0)
    r_ties = r1 - c_lt2

    tau24 = ((b1 - 2048) << 12) | b2
    vt = tau24 << 8
    v_rep = vt | 128

    sum_lt = _stage_c(jnp.full((1, 1), vt, jnp.int32), v_lo, v_hi)[0, 0]
    loss_neg = sum_lt + r_ties.astype(jnp.float32) * _ce_of_v(v_rep)
    loss_neg = jnp.where(k_eff > 0, loss_neg, jnp.float32(0.0))

    return (loss_pos + loss_neg) / (npos_f + n_neg.astype(jnp.float32))
